# Initial kernel scaffold; baseline (speedup 1.0000x reference)
import jax
import jax.numpy as jnp
from jax.experimental import pallas as pl


def kernel(x, W1, b1, W2, b2, batch):
    raise NotImplementedError("write your pallas kernel here")


# speedup vs baseline: 4.4900x; 4.4900x over previous
import jax
import jax.numpy as jnp
from jax.experimental import pallas as pl
from jax.experimental.pallas import tpu as pltpu


def _body(xb_ref, bt_ref, W1_ref, b1_ref, W2r_ref, b2_ref, out_ref,
          num_acc, den_acc, m_ref):
    i = pl.program_id(0)
    nb = pl.num_programs(0)
    R = xb_ref.shape[0]
    G = num_acc.shape[0]

    @pl.when(i == 0)
    def _init():
        num_acc[...] = jnp.zeros_like(num_acc)
        den_acc[...] = jnp.zeros_like(den_acc)
        m_ref[0, 0] = -jnp.inf

    xb = xb_ref[...]
    xb16 = xb.astype(jnp.bfloat16)
    w1 = W1_ref[...].astype(jnp.bfloat16)
    h = jnp.dot(xb16, w1, preferred_element_type=jnp.float32)
    h = jnp.maximum(h + b1_ref[...], 0.0)
    w = jnp.sum(h * W2r_ref[...], axis=1, keepdims=True) + b2_ref[0, 0]

    m_old = m_ref[0, 0]
    m_new = jnp.maximum(m_old, jnp.max(w))
    m_ref[0, 0] = m_new
    scale = jnp.exp(m_old - m_new)

    @pl.when(scale < 1.0)
    def _rescale():
        num_acc[...] = num_acc[...] * scale
        den_acc[...] = den_acc[...] * scale

    s = jnp.exp(w - m_new)
    gid = jax.lax.broadcasted_iota(jnp.int32, (R, G), 1)
    S = jnp.where(bt_ref[0] == gid, s, 0.0).astype(jnp.bfloat16)

    dn = (((0,), (0,)), ((), ()))
    num_acc[...] += jax.lax.dot_general(
        S, xb16, dn, preferred_element_type=jnp.float32)
    ones = jnp.ones((R, 128), jnp.bfloat16)
    den_acc[...] += jax.lax.dot_general(
        S, ones, dn, preferred_element_type=jnp.float32)

    @pl.when(i == nb - 1)
    def _finish():
        out_ref[...] = num_acc[...] / (den_acc[:, 0:1] + 1e-6)


def kernel(x, W1, b1, W2, b2, batch):
    N, D = x.shape
    H = W1.shape[1]
    G = 512
    R = 2000
    if N % R != 0:
        R = next(r for r in (1000, 500, 250, 200, 100, 50, 25, 20, 10, 8, N)
                 if N % r == 0)
    NB = N // R

    bt3 = batch.astype(jnp.int32).reshape(NB, R, 1)
    b1r = b1.reshape(1, H).astype(jnp.float32)
    W2r = W2.reshape(1, H).astype(jnp.float32)
    b2r = b2.reshape(1, 1).astype(jnp.float32)

    return pl.pallas_call(
        _body,
        grid=(NB,),
        in_specs=[
            pl.BlockSpec((R, D), lambda i: (i, 0)),
            pl.BlockSpec((1, R, 1), lambda i: (i, 0, 0)),
            pl.BlockSpec((D, H), lambda i: (0, 0)),
            pl.BlockSpec((1, H), lambda i: (0, 0)),
            pl.BlockSpec((1, H), lambda i: (0, 0)),
            pl.BlockSpec(memory_space=pltpu.SMEM),
        ],
        out_specs=pl.BlockSpec((G, D), lambda i: (0, 0)),
        out_shape=jax.ShapeDtypeStruct((G, D), jnp.float32),
        scratch_shapes=[
            pltpu.VMEM((G, D), jnp.float32),
            pltpu.VMEM((G, 128), jnp.float32),
            pltpu.SMEM((1, 1), jnp.float32),
        ],
        compiler_params=pltpu.CompilerParams(
            dimension_semantics=("arbitrary",)),
    )(x, bt3, W1, b1r, W2r, b2r)


# speedup vs baseline: 7.6959x; 1.7140x over previous
import jax
import jax.numpy as jnp
from jax.experimental import pallas as pl
from jax.experimental.pallas import tpu as pltpu


def _body(xb_ref, bt_ref, W1_ref, b1_ref, W2r_ref, b2_ref, out_ref,
          num_acc, den_acc, m_ref):
    i = pl.program_id(0)
    nb = pl.num_programs(0)
    R = xb_ref.shape[0]
    G = num_acc.shape[0]

    @pl.when(i == 0)
    def _init():
        num_acc[...] = jnp.zeros_like(num_acc)
        den_acc[...] = jnp.zeros_like(den_acc)
        m_ref[0, 0] = -jnp.inf

    xb = xb_ref[...]
    xb16 = xb.astype(jnp.bfloat16)
    w1 = W1_ref[...].astype(jnp.bfloat16)
    h = jnp.dot(xb16, w1, preferred_element_type=jnp.float32)
    h = jnp.maximum(h + b1_ref[...], 0.0)
    w_col = jnp.sum(h * W2r_ref[...], axis=1, keepdims=True) + b2_ref[0, 0]
    w_row = jnp.transpose(w_col)

    m_old = m_ref[0, 0]
    m_new = jnp.maximum(m_old, jnp.max(w_row))
    m_ref[0, 0] = m_new
    scale = jnp.exp(m_old - m_new)

    @pl.when(scale < 1.0)
    def _rescale():
        num_acc[...] = num_acc[...] * scale
        den_acc[...] = den_acc[...] * scale

    s_row = jnp.exp(w_row - m_new)
    gid = jax.lax.broadcasted_iota(jnp.int32, (G, R), 0)
    St = jnp.where(bt_ref[0] == gid, s_row, 0.0)
    den_acc[...] += jnp.sum(St, axis=1, keepdims=True)
    num_acc[...] += jnp.dot(St.astype(jnp.bfloat16), xb16,
                            preferred_element_type=jnp.float32)

    @pl.when(i == nb - 1)
    def _finish():
        out_ref[...] = num_acc[...] / (den_acc[...] + 1e-6)


def kernel(x, W1, b1, W2, b2, batch):
    N, D = x.shape
    H = W1.shape[1]
    G = 512
    R = 2000
    if N % R != 0:
        R = next(r for r in (1000, 500, 250, 200, 100, 50, 25, 20, 10, 8, N)
                 if N % r == 0)
    NB = N // R

    bt3 = batch.astype(jnp.int32).reshape(NB, 1, R)
    b1r = b1.reshape(1, H).astype(jnp.float32)
    W2r = W2.reshape(1, H).astype(jnp.float32)
    b2r = b2.reshape(1, 1).astype(jnp.float32)

    return pl.pallas_call(
        _body,
        grid=(NB,),
        in_specs=[
            pl.BlockSpec((R, D), lambda i: (i, 0)),
            pl.BlockSpec((1, 1, R), lambda i: (i, 0, 0)),
            pl.BlockSpec((D, H), lambda i: (0, 0)),
            pl.BlockSpec((1, H), lambda i: (0, 0)),
            pl.BlockSpec((1, H), lambda i: (0, 0)),
            pl.BlockSpec(memory_space=pltpu.SMEM),
        ],
        out_specs=pl.BlockSpec((G, D), lambda i: (0, 0)),
        out_shape=jax.ShapeDtypeStruct((G, D), jnp.float32),
        scratch_shapes=[
            pltpu.VMEM((G, D), jnp.float32),
            pltpu.VMEM((G, 1), jnp.float32),
            pltpu.SMEM((1, 1), jnp.float32),
        ],
        compiler_params=pltpu.CompilerParams(
            dimension_semantics=("arbitrary",)),
    )(x, bt3, W1, b1r, W2r, b2r)


# speedup vs baseline: 8.7211x; 1.1332x over previous
import jax
import jax.numpy as jnp
from jax.experimental import pallas as pl
from jax.experimental.pallas import tpu as pltpu


def _body(xb_ref, bt_ref, W1_ref, b1_ref, W2r_ref, b2_ref, out_ref,
          num_acc, den_acc, m_ref):
    i = pl.program_id(0)
    nb = pl.num_programs(0)
    R = xb_ref.shape[0]
    G = num_acc.shape[0]

    @pl.when(i == 0)
    def _init():
        num_acc[...] = jnp.zeros_like(num_acc)
        den_acc[...] = jnp.zeros_like(den_acc)
        m_ref[0, 0] = -jnp.inf

    xb = xb_ref[...]
    xb16 = xb.astype(jnp.bfloat16)
    w1 = W1_ref[...].astype(jnp.bfloat16)
    h = jnp.dot(xb16, w1, preferred_element_type=jnp.float32)
    h = jnp.maximum(h + b1_ref[...], 0.0)
    w_col = jnp.sum(h * W2r_ref[...], axis=1, keepdims=True) + b2_ref[0, 0]
    w_row = jnp.transpose(w_col)

    m_old = m_ref[0, 0]
    m_new = jnp.maximum(m_old, jnp.max(w_row))
    m_ref[0, 0] = m_new
    scale = jnp.exp(m_old - m_new)

    @pl.when(scale < 1.0)
    def _rescale():
        num_acc[...] = num_acc[...] * scale
        den_acc[...] = den_acc[...] * scale

    s_row = jnp.exp(w_row - m_new)
    gid = jax.lax.broadcasted_iota(jnp.int32, (G, R), 0)
    St = jnp.where(bt_ref[0] == gid, s_row, 0.0)
    den_acc[...] += jnp.sum(St, axis=1, keepdims=True)
    num_acc[...] += jnp.dot(St.astype(jnp.bfloat16), xb16,
                            preferred_element_type=jnp.float32)

    @pl.when(i == nb - 1)
    def _finish():
        out_ref[...] = num_acc[...] / (den_acc[...] + 1e-6)


def kernel(x, W1, b1, W2, b2, batch):
    N, D = x.shape
    H = W1.shape[1]
    G = 512
    R = 4000
    if N % R != 0:
        R = next(r for r in (2000, 1000, 500, 250, 200, 100, 50, 25, 20, 10,
                             8, N) if N % r == 0)
    NB = N // R

    bt3 = batch.astype(jnp.int32).reshape(NB, 1, R)
    b1r = b1.reshape(1, H).astype(jnp.float32)
    W2r = W2.reshape(1, H).astype(jnp.float32)
    b2r = b2.reshape(1, 1).astype(jnp.float32)

    return pl.pallas_call(
        _body,
        grid=(NB,),
        in_specs=[
            pl.BlockSpec((R, D), lambda i: (i, 0)),
            pl.BlockSpec((1, 1, R), lambda i: (i, 0, 0)),
            pl.BlockSpec((D, H), lambda i: (0, 0)),
            pl.BlockSpec((1, H), lambda i: (0, 0)),
            pl.BlockSpec((1, H), lambda i: (0, 0)),
            pl.BlockSpec(memory_space=pltpu.SMEM),
        ],
        out_specs=pl.BlockSpec((G, D), lambda i: (0, 0)),
        out_shape=jax.ShapeDtypeStruct((G, D), jnp.float32),
        scratch_shapes=[
            pltpu.VMEM((G, D), jnp.float32),
            pltpu.VMEM((G, 1), jnp.float32),
            pltpu.SMEM((1, 1), jnp.float32),
        ],
        compiler_params=pltpu.CompilerParams(
            dimension_semantics=("arbitrary",)),
    )(x, bt3, W1, b1r, W2r, b2r)


# speedup vs baseline: 10.5901x; 1.2143x over previous
import jax
import jax.numpy as jnp
from jax.experimental import pallas as pl
from jax.experimental.pallas import tpu as pltpu

_GW = 256


def _body(xb_ref, bt_ref, W1_ref, b1_ref, W2r_ref, b2_ref, lo_ref, ov_ref,
          out_ref, num_acc, den_acc, m_ref):
    i = pl.program_id(0)
    nb = pl.num_programs(0)
    R = xb_ref.shape[0]
    G = out_ref.shape[0]

    @pl.when(i == 0)
    def _init():
        num_acc[...] = jnp.zeros_like(num_acc)
        den_acc[...] = jnp.zeros_like(den_acc)
        m_ref[0, 0] = -jnp.inf

    xb = xb_ref[...]
    xb16 = xb.astype(jnp.bfloat16)
    w1 = W1_ref[...].astype(jnp.bfloat16)
    h = jnp.dot(xb16, w1, preferred_element_type=jnp.float32)
    h = jnp.maximum(h + b1_ref[...], 0.0)
    w_col = jnp.sum(h * W2r_ref[...], axis=1, keepdims=True) + b2_ref[0, 0]
    w_row = jnp.transpose(w_col)

    m_old = m_ref[0, 0]
    m_new = jnp.maximum(m_old, jnp.max(w_row))
    m_ref[0, 0] = m_new
    scale = jnp.exp(m_old - m_new)

    @pl.when(scale < 1.0)
    def _rescale():
        num_acc[...] = num_acc[...] * scale
        den_acc[...] = den_acc[...] * scale

    s_row = jnp.exp(w_row - m_new)
    bt = bt_ref[0]
    lo = pl.multiple_of(lo_ref[i], 8)

    @pl.when(ov_ref[i] == 0)
    def _narrow():
        gid = jax.lax.broadcasted_iota(jnp.int32, (_GW, R), 0)
        St = jnp.where(bt - lo == gid, s_row, 0.0)
        den_acc[pl.ds(lo, _GW), :] += jnp.sum(St, axis=1, keepdims=True)
        num_acc[pl.ds(lo, _GW), :] += jnp.dot(
            St.astype(jnp.bfloat16), xb16, preferred_element_type=jnp.float32)

    @pl.when(ov_ref[i] != 0)
    def _full():
        gid = jax.lax.broadcasted_iota(jnp.int32, (G, R), 0)
        St = jnp.where(bt == gid, s_row, 0.0)
        den_acc[:G, :] += jnp.sum(St, axis=1, keepdims=True)
        num_acc[:G, :] += jnp.dot(
            St.astype(jnp.bfloat16), xb16, preferred_element_type=jnp.float32)

    @pl.when(i == nb - 1)
    def _finish():
        out_ref[...] = num_acc[:G, :] / (den_acc[:G, :] + 1e-6)


def kernel(x, W1, b1, W2, b2, batch):
    N, D = x.shape
    H = W1.shape[1]
    G = 512
    R = 4000
    if N % R != 0:
        R = next(r for r in (2000, 1000, 500, 250, 200, 100, 50, 25, 20, 10,
                             8, N) if N % r == 0)
    NB = N // R

    batch32 = batch.astype(jnp.int32)
    bt3 = batch32.reshape(NB, 1, R)
    lo8 = (batch32[::R] // 8) * 8
    over = (batch32[R - 1::R] - lo8 >= _GW).astype(jnp.int32)
    b1r = b1.reshape(1, H).astype(jnp.float32)
    W2r = W2.reshape(1, H).astype(jnp.float32)
    b2r = b2.reshape(1, 1).astype(jnp.float32)

    return pl.pallas_call(
        _body,
        grid=(NB,),
        in_specs=[
            pl.BlockSpec((R, D), lambda i: (i, 0)),
            pl.BlockSpec((1, 1, R), lambda i: (i, 0, 0)),
            pl.BlockSpec((D, H), lambda i: (0, 0)),
            pl.BlockSpec((1, H), lambda i: (0, 0)),
            pl.BlockSpec((1, H), lambda i: (0, 0)),
            pl.BlockSpec(memory_space=pltpu.SMEM),
            pl.BlockSpec(memory_space=pltpu.SMEM),
            pl.BlockSpec(memory_space=pltpu.SMEM),
        ],
        out_specs=pl.BlockSpec((G, D), lambda i: (0, 0)),
        out_shape=jax.ShapeDtypeStruct((G, D), jnp.float32),
        scratch_shapes=[
            pltpu.VMEM((G + _GW, D), jnp.float32),
            pltpu.VMEM((G + _GW, 1), jnp.float32),
            pltpu.SMEM((1, 1), jnp.float32),
        ],
        compiler_params=pltpu.CompilerParams(
            dimension_semantics=("arbitrary",)),
    )(x, bt3, W1, b1r, W2r, b2r, lo8, over)


# speedup vs baseline: 11.7281x; 1.1075x over previous
import jax
import jax.numpy as jnp
from jax.experimental import pallas as pl
from jax.experimental.pallas import tpu as pltpu

_GW = 64


def _body(xb_ref, bt_ref, W1_ref, b1_ref, W2r_ref, b2_ref, lo_ref, ov_ref,
          out_ref, num_acc, den_acc, m_ref):
    i = pl.program_id(0)
    nb = pl.num_programs(0)
    R = xb_ref.shape[0]
    G = out_ref.shape[0]

    @pl.when(i == 0)
    def _init():
        num_acc[...] = jnp.zeros_like(num_acc)
        den_acc[...] = jnp.zeros_like(den_acc)
        m_ref[0, 0] = -jnp.inf

    xb = xb_ref[...]
    xb16 = xb.astype(jnp.bfloat16)
    w1 = W1_ref[...].astype(jnp.bfloat16)
    h = jnp.dot(xb16, w1, preferred_element_type=jnp.float32)
    h = jnp.maximum(h + b1_ref[...], 0.0)
    w_col = jnp.sum(h * W2r_ref[...], axis=1, keepdims=True) + b2_ref[0, 0]
    w_row = jnp.transpose(w_col)

    m_old = m_ref[0, 0]
    m_new = jnp.maximum(m_old, jnp.max(w_row))
    m_ref[0, 0] = m_new
    scale = jnp.exp(m_old - m_new)

    @pl.when(scale < 1.0)
    def _rescale():
        num_acc[...] = num_acc[...] * scale
        den_acc[...] = den_acc[...] * scale

    s_row = jnp.exp(w_row - m_new)
    bt = bt_ref[0]
    lo = pl.multiple_of(lo_ref[i], 8)

    @pl.when(ov_ref[i] == 0)
    def _narrow():
        gid = jax.lax.broadcasted_iota(jnp.int32, (_GW, R), 0)
        St = jnp.where(bt - lo == gid, s_row, 0.0)
        den_acc[pl.ds(lo, _GW), :] += jnp.sum(St, axis=1, keepdims=True)
        num_acc[pl.ds(lo, _GW), :] += jnp.dot(
            St.astype(jnp.bfloat16), xb16, preferred_element_type=jnp.float32)

    @pl.when(ov_ref[i] != 0)
    def _full():
        gid = jax.lax.broadcasted_iota(jnp.int32, (G, R), 0)
        St = jnp.where(bt == gid, s_row, 0.0)
        den_acc[:G, :] += jnp.sum(St, axis=1, keepdims=True)
        num_acc[:G, :] += jnp.dot(
            St.astype(jnp.bfloat16), xb16, preferred_element_type=jnp.float32)

    @pl.when(i == nb - 1)
    def _finish():
        out_ref[...] = num_acc[:G, :] / (den_acc[:G, :] + 1e-6)


def kernel(x, W1, b1, W2, b2, batch):
    N, D = x.shape
    H = W1.shape[1]
    G = 512
    R = 4000
    if N % R != 0:
        R = next(r for r in (2000, 1000, 500, 250, 200, 100, 50, 25, 20, 10,
                             8, N) if N % r == 0)
    NB = N // R

    batch32 = batch.astype(jnp.int32)
    bt3 = batch32.reshape(NB, 1, R)
    lo8 = (batch32[::R] // 8) * 8
    over = (batch32[R - 1::R] - lo8 >= _GW).astype(jnp.int32)
    b1r = b1.reshape(1, H).astype(jnp.float32)
    W2r = W2.reshape(1, H).astype(jnp.float32)
    b2r = b2.reshape(1, 1).astype(jnp.float32)

    return pl.pallas_call(
        _body,
        grid=(NB,),
        in_specs=[
            pl.BlockSpec((R, D), lambda i: (i, 0)),
            pl.BlockSpec((1, 1, R), lambda i: (i, 0, 0)),
            pl.BlockSpec((D, H), lambda i: (0, 0)),
            pl.BlockSpec((1, H), lambda i: (0, 0)),
            pl.BlockSpec((1, H), lambda i: (0, 0)),
            pl.BlockSpec(memory_space=pltpu.SMEM),
            pl.BlockSpec(memory_space=pltpu.SMEM),
            pl.BlockSpec(memory_space=pltpu.SMEM),
        ],
        out_specs=pl.BlockSpec((G, D), lambda i: (0, 0)),
        out_shape=jax.ShapeDtypeStruct((G, D), jnp.float32),
        scratch_shapes=[
            pltpu.VMEM((G + _GW, D), jnp.float32),
            pltpu.VMEM((G + _GW, 1), jnp.float32),
            pltpu.SMEM((1, 1), jnp.float32),
        ],
        compiler_params=pltpu.CompilerParams(
            dimension_semantics=("arbitrary",)),
    )(x, bt3, W1, b1r, W2r, b2r, lo8, over)
